# baseline (speedup 1.0000x reference)
import jax
import jax.numpy as jnp
from jax.experimental import pallas as pl
from jax.experimental.pallas import tpu as pltpu

_CODEBOOK = 1024
_D = 64
_BETA = 0.25


def _vq_body(z_ref, emb_ref, zq_ref, idx_ref, loss_ref):
    z = z_ref[0]
    emb = emb_ref[...]
    s = z.shape[1]

    ze = jax.lax.dot_general(
        emb, z, (((1,), (0,)), ((), ())),
        preferred_element_type=jnp.float32)
    z2 = jnp.sum(z * z, axis=0)
    e2 = jnp.sum(emb * emb, axis=1)
    dist = (z2[None, :] + e2[:, None]) - 2.0 * ze

    mind = jnp.min(dist, axis=0)
    iota = jax.lax.broadcasted_iota(jnp.int32, (_CODEBOOK, s), 0)
    big = jnp.int32(_CODEBOOK)
    idx = jnp.min(jnp.where(dist == mind[None, :], iota, big), axis=0)
    idx_ref[0, 0, :] = idx

    onehot = (iota == idx[None, :]).astype(jnp.float32)
    zq = jax.lax.dot_general(
        emb, onehot, (((0,), (0,)), ((), ())),
        precision=jax.lax.Precision.HIGHEST,
        preferred_element_type=jnp.float32)
    diff = zq - z
    zq_ref[0] = z + diff

    part = jnp.sum(diff * diff)

    @pl.when(pl.program_id(0) == 0)
    def _init():
        loss_ref[0, 0] = part

    @pl.when(pl.program_id(0) != 0)
    def _acc():
        loss_ref[0, 0] += part


def kernel(z_e, emb_weight):
    B, D, Gh, Gw = z_e.shape
    S = Gh * Gw
    z3 = z_e.reshape(B, D, S)

    zq3, idx3, loss_sum = pl.pallas_call(
        _vq_body,
        grid=(B,),
        in_specs=[
            pl.BlockSpec((1, D, S), lambda b: (b, 0, 0)),
            pl.BlockSpec((_CODEBOOK, D), lambda b: (0, 0)),
        ],
        out_specs=[
            pl.BlockSpec((1, D, S), lambda b: (b, 0, 0)),
            pl.BlockSpec((1, 1, S), lambda b: (b, 0, 0)),
            pl.BlockSpec((1, 1), lambda b: (0, 0), memory_space=pltpu.SMEM),
        ],
        out_shape=[
            jax.ShapeDtypeStruct((B, D, S), jnp.float32),
            jax.ShapeDtypeStruct((B, 1, S), jnp.int32),
            jax.ShapeDtypeStruct((1, 1), jnp.float32),
        ],
    )(z3, emb_weight)

    z_q_st = zq3.reshape(B, D, Gh, Gw)
    idx = idx3.reshape(B, Gh, Gw)
    mean_sq = loss_sum[0, 0] / jnp.float32(B * S * D)
    vq_loss = mean_sq + _BETA * mean_sq
    return (z_q_st, idx, vq_loss)


# speedup vs baseline: 1.6010x; 1.6010x over previous
import jax
import jax.numpy as jnp
from jax.experimental import pallas as pl
from jax.experimental.pallas import tpu as pltpu

_CODEBOOK = 1024
_D = 64
_BETA = 0.25


def _vq_body(z_ref, emb_ref, zq_ref, idx_ref, loss_ref):
    z = z_ref[0]
    emb = emb_ref[...]
    s = z.shape[1]

    ze = jax.lax.dot_general(
        emb, z, (((1,), (0,)), ((), ())),
        preferred_element_type=jnp.float32)
    z2 = jnp.sum(z * z, axis=0)
    e2 = jnp.sum(emb * emb, axis=1)
    dist = (z2[None, :] + e2[:, None]) - 2.0 * ze

    mind = jnp.min(dist, axis=0)
    iota = jax.lax.broadcasted_iota(jnp.int32, (_CODEBOOK, s), 0)
    big = jnp.int32(_CODEBOOK)
    idx = jnp.min(jnp.where(dist == mind[None, :], iota, big), axis=0)
    idx_ref[0, 0, :] = idx

    onehot = (iota == idx[None, :]).astype(jnp.float32)
    zq = jax.lax.dot_general(
        emb, onehot, (((0,), (0,)), ((), ())),
        preferred_element_type=jnp.float32)
    diff = zq - z
    zq_ref[0] = z + diff

    part = jnp.sum(diff * diff)

    @pl.when(pl.program_id(0) == 0)
    def _init():
        loss_ref[0, 0] = part

    @pl.when(pl.program_id(0) != 0)
    def _acc():
        loss_ref[0, 0] += part


def kernel(z_e, emb_weight):
    B, D, Gh, Gw = z_e.shape
    S = Gh * Gw
    z3 = z_e.reshape(B, D, S)

    zq3, idx3, loss_sum = pl.pallas_call(
        _vq_body,
        grid=(B,),
        in_specs=[
            pl.BlockSpec((1, D, S), lambda b: (b, 0, 0)),
            pl.BlockSpec((_CODEBOOK, D), lambda b: (0, 0)),
        ],
        out_specs=[
            pl.BlockSpec((1, D, S), lambda b: (b, 0, 0)),
            pl.BlockSpec((1, 1, S), lambda b: (b, 0, 0)),
            pl.BlockSpec((1, 1), lambda b: (0, 0), memory_space=pltpu.SMEM),
        ],
        out_shape=[
            jax.ShapeDtypeStruct((B, D, S), jnp.float32),
            jax.ShapeDtypeStruct((B, 1, S), jnp.int32),
            jax.ShapeDtypeStruct((1, 1), jnp.float32),
        ],
    )(z3, emb_weight)

    z_q_st = zq3.reshape(B, D, Gh, Gw)
    idx = idx3.reshape(B, Gh, Gw)
    mean_sq = loss_sum[0, 0] / jnp.float32(B * S * D)
    vq_loss = mean_sq + _BETA * mean_sq
    return (z_q_st, idx, vq_loss)


# speedup vs baseline: 1.9674x; 1.2288x over previous
import jax
import jax.numpy as jnp
from jax.experimental import pallas as pl
from jax.experimental.pallas import tpu as pltpu

_CODEBOOK = 1024
_D = 64
_BETA = 0.25
_BB = 2


def _vq_body(z_ref, emb_ref, zq_ref, idx_ref, loss_ref):
    z = jnp.concatenate([z_ref[i] for i in range(_BB)], axis=1)
    emb = emb_ref[...]
    s1 = z_ref.shape[2]
    s = _BB * s1

    ze = jax.lax.dot_general(
        emb, z, (((1,), (0,)), ((), ())),
        preferred_element_type=jnp.float32)
    z2 = jnp.sum(z * z, axis=0)
    e2 = jnp.sum(emb * emb, axis=1)
    dist = (z2[None, :] + e2[:, None]) - 2.0 * ze

    mind = jnp.min(dist, axis=0)
    iota = jax.lax.broadcasted_iota(jnp.int32, (_CODEBOOK, s), 0)
    big = jnp.int32(_CODEBOOK)
    idx = jnp.min(jnp.where(dist == mind[None, :], iota, big), axis=0)

    onehot = (iota == idx[None, :]).astype(jnp.float32)
    zq = jax.lax.dot_general(
        emb, onehot, (((0,), (0,)), ((), ())),
        preferred_element_type=jnp.float32)
    diff = zq - z
    out = z + diff
    for i in range(_BB):
        zq_ref[i] = out[:, i * s1:(i + 1) * s1]
        idx_ref[i, 0, :] = idx[i * s1:(i + 1) * s1]

    part = jnp.sum(diff * diff)

    @pl.when(pl.program_id(0) == 0)
    def _init():
        loss_ref[0, 0] = part

    @pl.when(pl.program_id(0) != 0)
    def _acc():
        loss_ref[0, 0] += part


def kernel(z_e, emb_weight):
    B, D, Gh, Gw = z_e.shape
    S = Gh * Gw
    z3 = z_e.reshape(B, D, S)

    zq3, idx3, loss_sum = pl.pallas_call(
        _vq_body,
        grid=(B // _BB,),
        in_specs=[
            pl.BlockSpec((_BB, D, S), lambda b: (b, 0, 0)),
            pl.BlockSpec((_CODEBOOK, D), lambda b: (0, 0)),
        ],
        out_specs=[
            pl.BlockSpec((_BB, D, S), lambda b: (b, 0, 0)),
            pl.BlockSpec((_BB, 1, S), lambda b: (b, 0, 0)),
            pl.BlockSpec((1, 1), lambda b: (0, 0), memory_space=pltpu.SMEM),
        ],
        out_shape=[
            jax.ShapeDtypeStruct((B, D, S), jnp.float32),
            jax.ShapeDtypeStruct((B, 1, S), jnp.int32),
            jax.ShapeDtypeStruct((1, 1), jnp.float32),
        ],
    )(z3, emb_weight)

    z_q_st = zq3.reshape(B, D, Gh, Gw)
    idx = idx3.reshape(B, Gh, Gw)
    mean_sq = loss_sum[0, 0] / jnp.float32(B * S * D)
    vq_loss = mean_sq + _BETA * mean_sq
    return (z_q_st, idx, vq_loss)


# speedup vs baseline: 2.0432x; 1.0385x over previous
import jax
import jax.numpy as jnp
from jax.experimental import pallas as pl
from jax.experimental.pallas import tpu as pltpu

_CODEBOOK = 1024
_D = 64
_BETA = 0.25
_BB = 4


def _vq_body(z_ref, emb_ref, zq_ref, idx_ref, loss_ref):
    z = jnp.concatenate([z_ref[i] for i in range(_BB)], axis=1)
    emb = emb_ref[...]
    s1 = z_ref.shape[2]
    s = _BB * s1

    ze = jax.lax.dot_general(
        emb, z, (((1,), (0,)), ((), ())),
        preferred_element_type=jnp.float32)
    z2 = jnp.sum(z * z, axis=0)
    e2 = jnp.sum(emb * emb, axis=1)
    dist = (z2[None, :] + e2[:, None]) - 2.0 * ze

    mind = jnp.min(dist, axis=0)
    iota = jax.lax.broadcasted_iota(jnp.int32, (_CODEBOOK, s), 0)
    big = jnp.int32(_CODEBOOK)
    idx = jnp.min(jnp.where(dist == mind[None, :], iota, big), axis=0)

    onehot = (iota == idx[None, :]).astype(jnp.float32)
    zq = jax.lax.dot_general(
        emb, onehot, (((0,), (0,)), ((), ())),
        preferred_element_type=jnp.float32)
    diff = zq - z
    out = z + diff
    for i in range(_BB):
        zq_ref[i] = out[:, i * s1:(i + 1) * s1]
        idx_ref[i, 0, :] = idx[i * s1:(i + 1) * s1]

    part = jnp.sum(diff * diff)

    @pl.when(pl.program_id(0) == 0)
    def _init():
        loss_ref[0, 0] = part

    @pl.when(pl.program_id(0) != 0)
    def _acc():
        loss_ref[0, 0] += part


def kernel(z_e, emb_weight):
    B, D, Gh, Gw = z_e.shape
    S = Gh * Gw
    z3 = z_e.reshape(B, D, S)

    zq3, idx3, loss_sum = pl.pallas_call(
        _vq_body,
        grid=(B // _BB,),
        in_specs=[
            pl.BlockSpec((_BB, D, S), lambda b: (b, 0, 0)),
            pl.BlockSpec((_CODEBOOK, D), lambda b: (0, 0)),
        ],
        out_specs=[
            pl.BlockSpec((_BB, D, S), lambda b: (b, 0, 0)),
            pl.BlockSpec((_BB, 1, S), lambda b: (b, 0, 0)),
            pl.BlockSpec((1, 1), lambda b: (0, 0), memory_space=pltpu.SMEM),
        ],
        out_shape=[
            jax.ShapeDtypeStruct((B, D, S), jnp.float32),
            jax.ShapeDtypeStruct((B, 1, S), jnp.int32),
            jax.ShapeDtypeStruct((1, 1), jnp.float32),
        ],
    )(z3, emb_weight)

    z_q_st = zq3.reshape(B, D, Gh, Gw)
    idx = idx3.reshape(B, Gh, Gw)
    mean_sq = loss_sum[0, 0] / jnp.float32(B * S * D)
    vq_loss = mean_sq + _BETA * mean_sq
    return (z_q_st, idx, vq_loss)
